# baseline (speedup 1.0000x reference)
import functools

import jax
import jax.numpy as jnp
from jax import lax
from jax.experimental import pallas as pl
from jax.experimental.pallas import tpu as pltpu
from jax.experimental.pallas import tpu_sc as plsc

B = 2
ROWS = 4096
LANES = 128
P = ROWS * LANES
CHR = 512
NCH = ROWS // CHR

K = 2048
NSLOT = 8
NBC = 2 * K
KROWS = K // LANES

SC_NC = 2
SC_NS = 16
SC_L = 16
NW = SC_NC * SC_NS
N_ELEM = NSLOT * P
SPAN = N_ELEM // NW
SC_CH = 16384
SC_NCHUNK = SPAN // SC_CH
HWORDS = NBC * SC_L

FMAX = 3.4028235e38
NSTAT = 14


def _xyz(ch):
    ri = lax.broadcasted_iota(jnp.int32, (CHR, LANES), 0) + ch * CHR
    li = lax.broadcasted_iota(jnp.int32, (CHR, LANES), 1)
    z = ri // 128
    y = ri - z * 128
    xm = li.astype(jnp.float32) * (1.0 / 127.0)
    ym = y.astype(jnp.float32) * (1.0 / 127.0)
    zm = z.astype(jnp.float32) * (1.0 / 31.0)
    return xm, ym, zm


def _p1_body(sig_ref, seed_ref, gt_ref, cl_ref, ce_ref, out_ref, acc):
    ch = pl.program_id(1)

    @pl.when(ch == 0)
    def _init():
        for j in range(64):
            acc[j] = 0.0

    xm, ym, zm = _xyz(ch)
    gt = gt_ref[0]
    ce_f = (ce_ref[0] != 0).astype(jnp.float32)
    cl_f = (cl_ref[0] != 0).astype(jnp.float32)
    seed = jax.nn.sigmoid(seed_ref[0])
    acc[56] = acc[56] + jnp.sum(seed * seed * (1.0 - cl_f))
    sx = sig_ref[0, 0]
    sy = sig_ref[0, 1]
    sz = sig_ref[0, 2]
    for i in range(4):
        mf = (gt == i + 1).astype(jnp.float32)
        cm = mf * ce_f
        o = i * NSTAT
        acc[o + 0] = acc[o + 0] + jnp.sum(mf)
        acc[o + 1] = acc[o + 1] + jnp.sum(cm)
        acc[o + 2] = acc[o + 2] + jnp.sum(xm * cm)
        acc[o + 3] = acc[o + 3] + jnp.sum(ym * cm)
        acc[o + 4] = acc[o + 4] + jnp.sum(zm * cm)
        acc[o + 5] = acc[o + 5] + jnp.sum(xm * mf)
        acc[o + 6] = acc[o + 6] + jnp.sum(ym * mf)
        acc[o + 7] = acc[o + 7] + jnp.sum(zm * mf)
        acc[o + 8] = acc[o + 8] + jnp.sum(sx * mf)
        acc[o + 9] = acc[o + 9] + jnp.sum(sy * mf)
        acc[o + 10] = acc[o + 10] + jnp.sum(sz * mf)
        acc[o + 11] = acc[o + 11] + jnp.sum(sx * sx * mf)
        acc[o + 12] = acc[o + 12] + jnp.sum(sy * sy * mf)
        acc[o + 13] = acc[o + 13] + jnp.sum(sz * sz * mf)

    @pl.when(ch == NCH - 1)
    def _flush():
        for j in range(64):
            out_ref[0, 0, j] = acc[j]


def _pass1(sig, seedr, gt, cl, ce):
    return pl.pallas_call(
        _p1_body,
        grid=(B, NCH),
        in_specs=[
            pl.BlockSpec((1, 3, CHR, LANES), lambda b, ch: (b, 0, ch, 0)),
            pl.BlockSpec((1, CHR, LANES), lambda b, ch: (b, ch, 0)),
            pl.BlockSpec((1, CHR, LANES), lambda b, ch: (b, ch, 0)),
            pl.BlockSpec((1, CHR, LANES), lambda b, ch: (b, ch, 0)),
            pl.BlockSpec((1, CHR, LANES), lambda b, ch: (b, ch, 0)),
        ],
        out_specs=pl.BlockSpec((1, 1, 64), lambda b, ch: (b, 0, 0),
                               memory_space=pltpu.SMEM),
        out_shape=jax.ShapeDtypeStruct((B, 1, 64), jnp.float32),
        scratch_shapes=[pltpu.SMEM((64,), jnp.float32)],
    )(sig, seedr, gt, cl, ce)


def _p2_body(emb_ref, seed_ref, gt_ref, stats_ref, bin_ref, sfg_ref, acc):
    ch = pl.program_id(1)

    @pl.when(ch == 0)
    def _init():
        for j in range(8):
            acc[j] = 0.0

    xm, ym, zm = _xyz(ch)
    ex = jnp.tanh(emb_ref[0, 0]) + xm
    ey = jnp.tanh(emb_ref[0, 1]) + ym
    ez = jnp.tanh(emb_ref[0, 2]) + zm
    seed = jax.nn.sigmoid(seed_ref[0])
    gt = gt_ref[0]
    kf = jnp.float32(K)
    for i in range(4):
        o = i * NSTAT
        cnt = stats_ref[0, 0, o + 0]
        ccnt = stats_ref[0, 0, o + 1]
        safe = jnp.maximum(cnt, 1.0)
        one_c = ccnt == 1.0
        cx = jnp.where(one_c, stats_ref[0, 0, o + 2], stats_ref[0, 0, o + 5] / safe)
        cy = jnp.where(one_c, stats_ref[0, 0, o + 3], stats_ref[0, 0, o + 6] / safe)
        cz = jnp.where(one_c, stats_ref[0, 0, o + 4], stats_ref[0, 0, o + 7] / safe)
        s4x = jnp.minimum(jnp.exp(10.0 * stats_ref[0, 0, o + 8] / safe), FMAX)
        s4y = jnp.minimum(jnp.exp(10.0 * stats_ref[0, 0, o + 9] / safe), FMAX)
        s4z = jnp.minimum(jnp.exp(10.0 * stats_ref[0, 0, o + 10] / safe), FMAX)
        q = ((ex - cx) * (ex - cx) * s4x + (ey - cy) * (ey - cy) * s4y
             + (ez - cz) * (ez - cz) * s4z)
        d = jnp.exp(-q)
        mi = gt == i + 1
        mf = mi.astype(jnp.float32)
        dv = seed - d
        acc[i] = acc[i] + jnp.sum(dv * dv * mf)
        binf = jnp.where(mi, kf - kf * d, kf * d)
        binn = jnp.clip(jnp.floor(binf).astype(jnp.int32), 0, K - 1)
        bin_ref[0, i] = binn + jnp.where(mi, K, 0)

    @pl.when(ch == NCH - 1)
    def _flush():
        for j in range(8):
            sfg_ref[0, 0, j] = acc[j]


def _pass2(emb, seedr, gt, stats):
    return pl.pallas_call(
        _p2_body,
        grid=(B, NCH),
        in_specs=[
            pl.BlockSpec((1, 3, CHR, LANES), lambda b, ch: (b, 0, ch, 0)),
            pl.BlockSpec((1, CHR, LANES), lambda b, ch: (b, ch, 0)),
            pl.BlockSpec((1, CHR, LANES), lambda b, ch: (b, ch, 0)),
            pl.BlockSpec((1, 1, 64), lambda b, ch: (b, 0, 0),
                         memory_space=pltpu.SMEM),
        ],
        out_specs=[
            pl.BlockSpec((1, 4, CHR, LANES), lambda b, ch: (b, 0, ch, 0)),
            pl.BlockSpec((1, 1, 8), lambda b, ch: (b, 0, 0),
                         memory_space=pltpu.SMEM),
        ],
        out_shape=[
            jax.ShapeDtypeStruct((B, 4, ROWS, LANES), jnp.int32),
            jax.ShapeDtypeStruct((B, 1, 8), jnp.float32),
        ],
        scratch_shapes=[pltpu.SMEM((8,), jnp.float32)],
    )(emb, seedr, gt, stats)


def _sc_hist_body(idx_hbm, out_hbm, histv, idxv):
    c = lax.axis_index("c")
    s = lax.axis_index("s")
    wid = s * SC_NC + c
    zeros16 = jnp.zeros((SC_L,), jnp.float32)
    ones16 = jnp.ones((SC_L,), jnp.float32)
    laneoff = lax.iota(jnp.int32, SC_L)

    def _zero(i, _):
        histv[pl.ds(i * SC_L, SC_L)] = zeros16
        return 0
    lax.fori_loop(0, HWORDS // SC_L, _zero, 0, unroll=8)

    base = wid * SPAN
    for j in range(SC_NCHUNK):
        pltpu.sync_copy(idx_hbm.at[pl.ds(base + j * SC_CH, SC_CH)], idxv)

        def _scat(i, _):
            v = idxv[pl.ds(i * SC_L, SC_L)]
            plsc.addupdate_scatter(histv, [v * SC_L + laneoff], ones16)
            return 0
        lax.fori_loop(0, SC_CH // SC_L, _scat, 0, unroll=8)

    pltpu.sync_copy(histv, out_hbm.at[wid])


def _sc_histogram(idx_flat):
    mesh = plsc.VectorSubcoreMesh(core_axis_name="c", subcore_axis_name="s")
    kern = functools.partial(
        pl.kernel,
        mesh=mesh,
        compiler_params=pltpu.CompilerParams(needs_layout_passes=False),
        out_type=jax.ShapeDtypeStruct((NW, HWORDS), jnp.float32),
        scratch_types=[
            pltpu.VMEM((HWORDS,), jnp.float32),
            pltpu.VMEM((SC_CH,), jnp.int32),
        ],
    )(_sc_hist_body)
    return kern(idx_flat)


HR = HWORDS // LANES
GR = HR // 2


def _fin_body(hist_ref, stats_ref, sfg_ref, out_ref):
    ci = lax.broadcasted_iota(jnp.int32, (LANES, 8), 0)
    gi = lax.broadcasted_iota(jnp.int32, (LANES, 8), 1)
    s_red = (ci // SC_L == gi).astype(jnp.float32)
    k8 = lax.broadcasted_iota(jnp.int32, (8, 8), 0)
    c8 = lax.broadcasted_iota(jnp.int32, (8, 8), 1)
    t_suf = (k8 >= c8).astype(jnp.float32)
    kr = lax.broadcasted_iota(jnp.int32, (GR, GR), 0)
    cr = lax.broadcasted_iota(jnp.int32, (GR, GR), 1)
    a_suf = (cr > kr).astype(jnp.float32)
    wbin = jnp.float32(2.0 / K)
    total = jnp.float32(0.0)
    for b in range(B):
        inst = jnp.float32(0.0)
        var = jnp.float32(0.0)
        seedl = stats_ref[b, 0, 56]
        obj = jnp.float32(0.0)
        for i in range(4):
            sl = b * 4 + i
            slh4 = (hist_ref[4 * sl] + hist_ref[4 * sl + 1]
                    + hist_ref[4 * sl + 2] + hist_ref[4 * sl + 3])
            g_all = jnp.dot(slh4, s_red, preferred_element_type=jnp.float32)
            n0 = g_all[0:GR, :]
            n1 = g_all[GR:2 * GR, :]
            tot = n0 + n1
            wc = jnp.dot(tot, t_suf, preferred_element_type=jnp.float32)
            wg = jnp.dot(n1, t_suf, preferred_element_type=jnp.float32)
            rc = jnp.dot(a_suf, wc[:, 0:1], preferred_element_type=jnp.float32)
            rg = jnp.dot(a_suf, wg[:, 0:1], preferred_element_type=jnp.float32)
            sc_ = wc + rc
            sg = wg + rg
            o = i * NSTAT
            cnt = stats_ref[b, 0, o + 0]
            present = (cnt > 0.0).astype(jnp.float32)
            safe = jnp.maximum(cnt, 1.0)
            g_tot = cnt
            h = (g_tot - sg) / jnp.maximum(g_tot + sc_ - sg, 1.0)
            h_k = g_tot / jnp.maximum(g_tot, 1.0)
            hsum = jnp.sum(h) + h_k
            s_lov = 2.0 - wbin * (hsum - 0.5 * h_k)
            inst = inst + present * s_lov
            vs = jnp.float32(0.0)
            for k in range(3):
                ssum = stats_ref[b, 0, o + 8 + k]
                s2sum = stats_ref[b, 0, o + 11 + k]
                sm = ssum / safe
                vs = vs + (s2sum - 2.0 * sm * ssum + sm * sm * cnt)
            var = var + present * vs / (3.0 * safe)
            seedl = seedl + present * 10.0 * sfg_ref[b, 0, i]
            obj = obj + present
        so = jnp.maximum(obj, 1.0)
        total = total + inst / so + 10.0 * var / so + seedl / jnp.float32(P)
    out_ref[0] = total * jnp.float32(1.0 / B)


def _finalize(hist, stats, sfg):
    return pl.pallas_call(
        _fin_body,
        grid=(1,),
        in_specs=[
            pl.BlockSpec((NW, HR, LANES), lambda _: (0, 0, 0)),
            pl.BlockSpec((B, 1, 64), lambda _: (0, 0, 0),
                         memory_space=pltpu.SMEM),
            pl.BlockSpec((B, 1, 8), lambda _: (0, 0, 0),
                         memory_space=pltpu.SMEM),
        ],
        out_specs=pl.BlockSpec((1,), lambda _: (0,), memory_space=pltpu.SMEM),
        out_shape=jax.ShapeDtypeStruct((1,), jnp.float32),
    )(hist, stats, sfg)


def kernel(prediction, GT, CL, CE):
    emb = prediction[:, 0:3].reshape(B, 3, ROWS, LANES)
    sig = prediction[:, 3:6].reshape(B, 3, ROWS, LANES)
    seedr = prediction[:, 6].reshape(B, ROWS, LANES)
    gt = GT.reshape(B, ROWS, LANES).astype(jnp.int32)
    cl = CL.reshape(B, ROWS, LANES).astype(jnp.int32)
    ce = CE.reshape(B, ROWS, LANES).astype(jnp.int32)

    stats = _pass1(sig, seedr, gt, cl, ce)
    binidx, sfg = _pass2(emb, seedr, gt, stats)
    hist = _sc_histogram(binidx.reshape(N_ELEM))
    out = _finalize(hist.reshape(NW, HWORDS // LANES, LANES), stats, sfg)
    return out[0]


# speedup vs baseline: 1.2529x; 1.2529x over previous
import functools

import jax
import jax.numpy as jnp
from jax import lax
from jax.experimental import pallas as pl
from jax.experimental.pallas import tpu as pltpu
from jax.experimental.pallas import tpu_sc as plsc

B = 2
ROWS = 4096
LANES = 128
P = ROWS * LANES
CHR = 512
NCH = ROWS // CHR

K = 2048
NSLOT = 8
NBC = 2 * K
KROWS = K // LANES

SC_NC = 2
SC_NS = 16
SC_L = 16
NW = SC_NC * SC_NS
N_ELEM = NSLOT * P
SPAN = N_ELEM // NW
SC_CH = 16384
SC_NCHUNK = SPAN // SC_CH
NB = NSLOT * NBC
STRIPE = NB // SC_NS

FMAX = 3.4028235e38
NSTAT = 14


def _xyz(ch):
    ri = lax.broadcasted_iota(jnp.int32, (CHR, LANES), 0) + ch * CHR
    li = lax.broadcasted_iota(jnp.int32, (CHR, LANES), 1)
    z = ri // 128
    y = ri - z * 128
    xm = li.astype(jnp.float32) * (1.0 / 127.0)
    ym = y.astype(jnp.float32) * (1.0 / 127.0)
    zm = z.astype(jnp.float32) * (1.0 / 31.0)
    return xm, ym, zm


def _p1_body(sig_ref, seed_ref, gt_ref, cl_ref, ce_ref, out_ref, acc):
    ch = pl.program_id(1)

    @pl.when(ch == 0)
    def _init():
        for j in range(64):
            acc[j] = 0.0

    xm, ym, zm = _xyz(ch)
    gt = gt_ref[0]
    ce_f = (ce_ref[0] != 0).astype(jnp.float32)
    cl_f = (cl_ref[0] != 0).astype(jnp.float32)
    seed = jax.nn.sigmoid(seed_ref[0])
    acc[56] = acc[56] + jnp.sum(seed * seed * (1.0 - cl_f))
    sx = sig_ref[0, 0]
    sy = sig_ref[0, 1]
    sz = sig_ref[0, 2]
    for i in range(4):
        mf = (gt == i + 1).astype(jnp.float32)
        cm = mf * ce_f
        o = i * NSTAT
        acc[o + 0] = acc[o + 0] + jnp.sum(mf)
        acc[o + 1] = acc[o + 1] + jnp.sum(cm)
        acc[o + 2] = acc[o + 2] + jnp.sum(xm * cm)
        acc[o + 3] = acc[o + 3] + jnp.sum(ym * cm)
        acc[o + 4] = acc[o + 4] + jnp.sum(zm * cm)
        acc[o + 5] = acc[o + 5] + jnp.sum(xm * mf)
        acc[o + 6] = acc[o + 6] + jnp.sum(ym * mf)
        acc[o + 7] = acc[o + 7] + jnp.sum(zm * mf)
        acc[o + 8] = acc[o + 8] + jnp.sum(sx * mf)
        acc[o + 9] = acc[o + 9] + jnp.sum(sy * mf)
        acc[o + 10] = acc[o + 10] + jnp.sum(sz * mf)
        acc[o + 11] = acc[o + 11] + jnp.sum(sx * sx * mf)
        acc[o + 12] = acc[o + 12] + jnp.sum(sy * sy * mf)
        acc[o + 13] = acc[o + 13] + jnp.sum(sz * sz * mf)

    @pl.when(ch == NCH - 1)
    def _flush():
        for j in range(64):
            out_ref[0, 0, j] = acc[j]


def _pass1(sig, seedr, gt, cl, ce):
    return pl.pallas_call(
        _p1_body,
        grid=(B, NCH),
        in_specs=[
            pl.BlockSpec((1, 3, CHR, LANES), lambda b, ch: (b, 0, ch, 0)),
            pl.BlockSpec((1, CHR, LANES), lambda b, ch: (b, ch, 0)),
            pl.BlockSpec((1, CHR, LANES), lambda b, ch: (b, ch, 0)),
            pl.BlockSpec((1, CHR, LANES), lambda b, ch: (b, ch, 0)),
            pl.BlockSpec((1, CHR, LANES), lambda b, ch: (b, ch, 0)),
        ],
        out_specs=pl.BlockSpec((1, 1, 64), lambda b, ch: (b, 0, 0),
                               memory_space=pltpu.SMEM),
        out_shape=jax.ShapeDtypeStruct((B, 1, 64), jnp.float32),
        scratch_shapes=[pltpu.SMEM((64,), jnp.float32)],
    )(sig, seedr, gt, cl, ce)


def _p2_body(emb_ref, seed_ref, gt_ref, stats_ref, bin_ref, sfg_ref, acc):
    b = pl.program_id(0)
    ch = pl.program_id(1)

    @pl.when(ch == 0)
    def _init():
        for j in range(8):
            acc[j] = 0.0

    xm, ym, zm = _xyz(ch)
    ex = jnp.tanh(emb_ref[0, 0]) + xm
    ey = jnp.tanh(emb_ref[0, 1]) + ym
    ez = jnp.tanh(emb_ref[0, 2]) + zm
    seed = jax.nn.sigmoid(seed_ref[0])
    gt = gt_ref[0]
    kf = jnp.float32(K)
    for i in range(4):
        o = i * NSTAT
        cnt = stats_ref[0, 0, o + 0]
        ccnt = stats_ref[0, 0, o + 1]
        safe = jnp.maximum(cnt, 1.0)
        one_c = ccnt == 1.0
        cx = jnp.where(one_c, stats_ref[0, 0, o + 2], stats_ref[0, 0, o + 5] / safe)
        cy = jnp.where(one_c, stats_ref[0, 0, o + 3], stats_ref[0, 0, o + 6] / safe)
        cz = jnp.where(one_c, stats_ref[0, 0, o + 4], stats_ref[0, 0, o + 7] / safe)
        s4x = jnp.minimum(jnp.exp(10.0 * stats_ref[0, 0, o + 8] / safe), FMAX)
        s4y = jnp.minimum(jnp.exp(10.0 * stats_ref[0, 0, o + 9] / safe), FMAX)
        s4z = jnp.minimum(jnp.exp(10.0 * stats_ref[0, 0, o + 10] / safe), FMAX)
        q = ((ex - cx) * (ex - cx) * s4x + (ey - cy) * (ey - cy) * s4y
             + (ez - cz) * (ez - cz) * s4z)
        d = jnp.exp(-q)
        mi = gt == i + 1
        mf = mi.astype(jnp.float32)
        dv = seed - d
        acc[i] = acc[i] + jnp.sum(dv * dv * mf)
        binf = jnp.where(mi, kf - kf * d, kf * d)
        binn = jnp.clip(jnp.floor(binf).astype(jnp.int32), 0, K - 1)
        slotbase = ((b * 4 + i) * 2) * K
        bin_ref[0, i] = binn + jnp.where(mi, slotbase + K, slotbase)

    @pl.when(ch == NCH - 1)
    def _flush():
        for j in range(8):
            sfg_ref[0, 0, j] = acc[j]


def _pass2(emb, seedr, gt, stats):
    return pl.pallas_call(
        _p2_body,
        grid=(B, NCH),
        in_specs=[
            pl.BlockSpec((1, 3, CHR, LANES), lambda b, ch: (b, 0, ch, 0)),
            pl.BlockSpec((1, CHR, LANES), lambda b, ch: (b, ch, 0)),
            pl.BlockSpec((1, CHR, LANES), lambda b, ch: (b, ch, 0)),
            pl.BlockSpec((1, 1, 64), lambda b, ch: (b, 0, 0),
                         memory_space=pltpu.SMEM),
        ],
        out_specs=[
            pl.BlockSpec((1, 4, CHR, LANES), lambda b, ch: (b, 0, ch, 0)),
            pl.BlockSpec((1, 1, 8), lambda b, ch: (b, 0, 0),
                         memory_space=pltpu.SMEM),
        ],
        out_shape=[
            jax.ShapeDtypeStruct((B, 4, ROWS, LANES), jnp.int32),
            jax.ShapeDtypeStruct((B, 1, 8), jnp.float32),
        ],
        scratch_shapes=[pltpu.SMEM((8,), jnp.float32)],
    )(emb, seedr, gt, stats)


def _sc_hist_body(idx_hbm, zeros_hbm, ones_hbm, out_hbm,
                  hist_sh, idxv0, idxv1, onesv, stagev, sem0, sem1):
    c = lax.axis_index("c")
    s = lax.axis_index("s")
    wid = s * SC_NC + c
    pltpu.sync_copy(zeros_hbm, stagev)
    pltpu.sync_copy(stagev, hist_sh.at[pl.ds(s * STRIPE, STRIPE)])
    pltpu.sync_copy(ones_hbm, onesv)
    plsc.subcore_barrier()
    base = wid * SPAN
    bufs = (idxv0, idxv1)
    sems = (sem0, sem1)
    cps = [None, None]
    cps[0] = pltpu.async_copy(idx_hbm.at[pl.ds(base, SC_CH)], idxv0, sem0)
    for j in range(SC_NCHUNK):
        nxt = (j + 1) % 2
        if j + 1 < SC_NCHUNK:
            cps[nxt] = pltpu.async_copy(
                idx_hbm.at[pl.ds(base + (j + 1) * SC_CH, SC_CH)],
                bufs[nxt], sems[nxt])
        cps[j % 2].wait()
        pltpu.sync_copy(onesv, hist_sh.at[bufs[j % 2]], add=True)
    plsc.subcore_barrier()
    pltpu.sync_copy(hist_sh.at[pl.ds(s * STRIPE, STRIPE)], stagev)
    pltpu.sync_copy(stagev, out_hbm.at[c, pl.ds(s * STRIPE, STRIPE)])


def _sc_histogram(idx_flat, zeros, ones):
    mesh = plsc.VectorSubcoreMesh(core_axis_name="c", subcore_axis_name="s")
    kern = functools.partial(
        pl.kernel,
        mesh=mesh,
        compiler_params=pltpu.CompilerParams(needs_layout_passes=False),
        out_type=jax.ShapeDtypeStruct((SC_NC, NB), jnp.float32),
        scratch_types=[
            pltpu.VMEM_SHARED((NB,), jnp.float32),
            pltpu.VMEM((SC_CH,), jnp.int32),
            pltpu.VMEM((SC_CH,), jnp.int32),
            pltpu.VMEM((SC_CH,), jnp.float32),
            pltpu.VMEM((STRIPE,), jnp.float32),
            pltpu.SemaphoreType.DMA,
            pltpu.SemaphoreType.DMA,
        ],
    )(_sc_hist_body)
    return kern(idx_flat, zeros, ones)


def _fin_body(hist_ref, stats_ref, sfg_ref, out_ref):
    hist = hist_ref[0] + hist_ref[1]
    ki = lax.broadcasted_iota(jnp.int32, (LANES, LANES), 0)
    ci = lax.broadcasted_iota(jnp.int32, (LANES, LANES), 1)
    t_suf = (ki >= ci).astype(jnp.float32)
    kr = lax.broadcasted_iota(jnp.int32, (KROWS, KROWS), 0)
    cr = lax.broadcasted_iota(jnp.int32, (KROWS, KROWS), 1)
    a_suf = (cr > kr).astype(jnp.float32)
    wbin = jnp.float32(2.0 / K)
    total = jnp.float32(0.0)
    for b in range(B):
        inst = jnp.float32(0.0)
        var = jnp.float32(0.0)
        seedl = stats_ref[b, 0, 56]
        obj = jnp.float32(0.0)
        for i in range(4):
            sl = b * 4 + i
            r0 = sl * 2 * KROWS
            n0 = hist[r0:r0 + KROWS, :]
            n1 = hist[r0 + KROWS:r0 + 2 * KROWS, :]
            tot = n0 + n1
            wc = jnp.dot(tot, t_suf, preferred_element_type=jnp.float32)
            wg = jnp.dot(n1, t_suf, preferred_element_type=jnp.float32)
            rc = jnp.dot(a_suf, wc[:, 0:1], preferred_element_type=jnp.float32)
            rg = jnp.dot(a_suf, wg[:, 0:1], preferred_element_type=jnp.float32)
            sc_ = wc + rc
            sg = wg + rg
            o = i * NSTAT
            cnt = stats_ref[b, 0, o + 0]
            present = (cnt > 0.0).astype(jnp.float32)
            safe = jnp.maximum(cnt, 1.0)
            g_tot = cnt
            h = (g_tot - sg) / jnp.maximum(g_tot + sc_ - sg, 1.0)
            h_k = g_tot / jnp.maximum(g_tot, 1.0)
            hsum = jnp.sum(h) + h_k
            s_lov = 2.0 - wbin * (hsum - 0.5 * h_k)
            inst = inst + present * s_lov
            vs = jnp.float32(0.0)
            for k in range(3):
                ssum = stats_ref[b, 0, o + 8 + k]
                s2sum = stats_ref[b, 0, o + 11 + k]
                sm = ssum / safe
                vs = vs + (s2sum - 2.0 * sm * ssum + sm * sm * cnt)
            var = var + present * vs / (3.0 * safe)
            seedl = seedl + present * 10.0 * sfg_ref[b, 0, i]
            obj = obj + present
        so = jnp.maximum(obj, 1.0)
        total = total + inst / so + 10.0 * var / so + seedl / jnp.float32(P)
    out_ref[0] = total * jnp.float32(1.0 / B)


def _finalize(hist, stats, sfg):
    return pl.pallas_call(
        _fin_body,
        grid=(1,),
        in_specs=[
            pl.BlockSpec((SC_NC, NB // LANES, LANES), lambda _: (0, 0, 0)),
            pl.BlockSpec((B, 1, 64), lambda _: (0, 0, 0),
                         memory_space=pltpu.SMEM),
            pl.BlockSpec((B, 1, 8), lambda _: (0, 0, 0),
                         memory_space=pltpu.SMEM),
        ],
        out_specs=pl.BlockSpec((1,), lambda _: (0,), memory_space=pltpu.SMEM),
        out_shape=jax.ShapeDtypeStruct((1,), jnp.float32),
    )(hist, stats, sfg)


def kernel(prediction, GT, CL, CE):
    emb = prediction[:, 0:3].reshape(B, 3, ROWS, LANES)
    sig = prediction[:, 3:6].reshape(B, 3, ROWS, LANES)
    seedr = prediction[:, 6].reshape(B, ROWS, LANES)
    gt = GT.reshape(B, ROWS, LANES).astype(jnp.int32)
    cl = CL.reshape(B, ROWS, LANES).astype(jnp.int32)
    ce = CE.reshape(B, ROWS, LANES).astype(jnp.int32)

    stats = _pass1(sig, seedr, gt, cl, ce)
    binidx, sfg = _pass2(emb, seedr, gt, stats)
    zeros = jnp.zeros((STRIPE,), jnp.float32)
    ones = jnp.ones((SC_CH,), jnp.float32)
    hist = _sc_histogram(binidx.reshape(N_ELEM), zeros, ones)
    out = _finalize(hist.reshape(SC_NC, NB // LANES, LANES), stats, sfg)
    return out[0]


# speedup vs baseline: 1.4460x; 1.1541x over previous
import functools

import jax
import jax.numpy as jnp
from jax import lax
from jax.experimental import pallas as pl
from jax.experimental.pallas import tpu as pltpu
from jax.experimental.pallas import tpu_sc as plsc

B = 2
ROWS = 4096
LANES = 128
P = ROWS * LANES
CHR = 512
NCH = ROWS // CHR

K = 2048
NSLOT = 8
NBC = 2 * K
KROWS = K // LANES

SC_NC = 2
SC_NS = 16
SC_L = 16
NW = SC_NC * SC_NS
N_ELEM = NSLOT * P
SPAN = N_ELEM // NW
SC_CH = 16384
SC_NCHUNK = SPAN // SC_CH
NB = NSLOT * NBC
STRIPE = NB // SC_NS

FMAX = 3.4028235e38
NSTAT = 14


def _xyz(ch):
    ri = lax.broadcasted_iota(jnp.int32, (CHR, LANES), 0) + ch * CHR
    li = lax.broadcasted_iota(jnp.int32, (CHR, LANES), 1)
    z = ri // 128
    y = ri - z * 128
    xm = li.astype(jnp.float32) * (1.0 / 127.0)
    ym = y.astype(jnp.float32) * (1.0 / 127.0)
    zm = z.astype(jnp.float32) * (1.0 / 31.0)
    return xm, ym, zm


def _p1_body(sig_ref, seed_ref, gt_ref, cl_ref, ce_ref, out_ref, acc):
    ch = pl.program_id(1)

    @pl.when(ch == 0)
    def _init():
        for j in range(64):
            acc[j] = 0.0

    xm, ym, zm = _xyz(ch)
    gt = gt_ref[0]
    ce_f = (ce_ref[0] != 0).astype(jnp.float32)
    cl_f = (cl_ref[0] != 0).astype(jnp.float32)
    seed = jax.nn.sigmoid(seed_ref[0, 0])
    acc[56] = acc[56] + jnp.sum(seed * seed * (1.0 - cl_f))
    sx = sig_ref[0, 0]
    sy = sig_ref[0, 1]
    sz = sig_ref[0, 2]
    for i in range(4):
        mf = (gt == i + 1).astype(jnp.float32)
        cm = mf * ce_f
        o = i * NSTAT
        acc[o + 0] = acc[o + 0] + jnp.sum(mf)
        acc[o + 1] = acc[o + 1] + jnp.sum(cm)
        acc[o + 2] = acc[o + 2] + jnp.sum(xm * cm)
        acc[o + 3] = acc[o + 3] + jnp.sum(ym * cm)
        acc[o + 4] = acc[o + 4] + jnp.sum(zm * cm)
        acc[o + 5] = acc[o + 5] + jnp.sum(xm * mf)
        acc[o + 6] = acc[o + 6] + jnp.sum(ym * mf)
        acc[o + 7] = acc[o + 7] + jnp.sum(zm * mf)
        acc[o + 8] = acc[o + 8] + jnp.sum(sx * mf)
        acc[o + 9] = acc[o + 9] + jnp.sum(sy * mf)
        acc[o + 10] = acc[o + 10] + jnp.sum(sz * mf)
        acc[o + 11] = acc[o + 11] + jnp.sum(sx * sx * mf)
        acc[o + 12] = acc[o + 12] + jnp.sum(sy * sy * mf)
        acc[o + 13] = acc[o + 13] + jnp.sum(sz * sz * mf)

    @pl.when(ch == NCH - 1)
    def _flush():
        for j in range(64):
            out_ref[0, 0, j] = acc[j]


def _pass1(pred, gt, cl, ce):
    return pl.pallas_call(
        _p1_body,
        grid=(B, NCH),
        in_specs=[
            pl.BlockSpec((1, 3, CHR, LANES), lambda b, ch: (b, 1, ch, 0)),
            pl.BlockSpec((1, 1, CHR, LANES), lambda b, ch: (b, 6, ch, 0)),
            pl.BlockSpec((1, CHR, LANES), lambda b, ch: (b, ch, 0)),
            pl.BlockSpec((1, CHR, LANES), lambda b, ch: (b, ch, 0)),
            pl.BlockSpec((1, CHR, LANES), lambda b, ch: (b, ch, 0)),
        ],
        out_specs=pl.BlockSpec((1, 1, 64), lambda b, ch: (b, 0, 0),
                               memory_space=pltpu.SMEM),
        out_shape=jax.ShapeDtypeStruct((B, 1, 64), jnp.float32),
        scratch_shapes=[pltpu.SMEM((64,), jnp.float32)],
    )(pred, pred, gt, cl, ce)


def _p2_body(emb_ref, seed_ref, gt_ref, stats_ref, bin_ref, sfg_ref, acc):
    b = pl.program_id(0)
    ch = pl.program_id(1)

    @pl.when(ch == 0)
    def _init():
        for j in range(8):
            acc[j] = 0.0

    xm, ym, zm = _xyz(ch)
    ex = jnp.tanh(emb_ref[0, 0]) + xm
    ey = jnp.tanh(emb_ref[0, 1]) + ym
    ez = jnp.tanh(emb_ref[0, 2]) + zm
    seed = jax.nn.sigmoid(seed_ref[0, 0])
    gt = gt_ref[0]
    kf = jnp.float32(K)
    for i in range(4):
        o = i * NSTAT
        cnt = stats_ref[0, 0, o + 0]
        ccnt = stats_ref[0, 0, o + 1]
        safe = jnp.maximum(cnt, 1.0)
        one_c = ccnt == 1.0
        cx = jnp.where(one_c, stats_ref[0, 0, o + 2], stats_ref[0, 0, o + 5] / safe)
        cy = jnp.where(one_c, stats_ref[0, 0, o + 3], stats_ref[0, 0, o + 6] / safe)
        cz = jnp.where(one_c, stats_ref[0, 0, o + 4], stats_ref[0, 0, o + 7] / safe)
        s4x = jnp.minimum(jnp.exp(10.0 * stats_ref[0, 0, o + 8] / safe), FMAX)
        s4y = jnp.minimum(jnp.exp(10.0 * stats_ref[0, 0, o + 9] / safe), FMAX)
        s4z = jnp.minimum(jnp.exp(10.0 * stats_ref[0, 0, o + 10] / safe), FMAX)
        q = ((ex - cx) * (ex - cx) * s4x + (ey - cy) * (ey - cy) * s4y
             + (ez - cz) * (ez - cz) * s4z)
        d = jnp.exp(-q)
        mi = gt == i + 1
        mf = mi.astype(jnp.float32)
        dv = seed - d
        acc[i] = acc[i] + jnp.sum(dv * dv * mf)
        binf = jnp.where(mi, kf - kf * d, kf * d)
        binn = jnp.clip(jnp.floor(binf).astype(jnp.int32), 0, K - 1)
        slotbase = ((b * 4 + i) * 2) * K
        bin_ref[0, i] = binn + jnp.where(mi, slotbase + K, slotbase)

    @pl.when(ch == NCH - 1)
    def _flush():
        for j in range(8):
            sfg_ref[0, 0, j] = acc[j]


def _pass2(pred, gt, stats):
    return pl.pallas_call(
        _p2_body,
        grid=(B, NCH),
        in_specs=[
            pl.BlockSpec((1, 3, CHR, LANES), lambda b, ch: (b, 0, ch, 0)),
            pl.BlockSpec((1, 1, CHR, LANES), lambda b, ch: (b, 6, ch, 0)),
            pl.BlockSpec((1, CHR, LANES), lambda b, ch: (b, ch, 0)),
            pl.BlockSpec((1, 1, 64), lambda b, ch: (b, 0, 0),
                         memory_space=pltpu.SMEM),
        ],
        out_specs=[
            pl.BlockSpec((1, 4, CHR, LANES), lambda b, ch: (b, 0, ch, 0)),
            pl.BlockSpec((1, 1, 8), lambda b, ch: (b, 0, 0),
                         memory_space=pltpu.SMEM),
        ],
        out_shape=[
            jax.ShapeDtypeStruct((B, 4, ROWS, LANES), jnp.int32),
            jax.ShapeDtypeStruct((B, 1, 8), jnp.float32),
        ],
        scratch_shapes=[pltpu.SMEM((8,), jnp.float32)],
    )(pred, pred, gt, stats)


def _sc_hist_body(idx_hbm, zeros_hbm, ones_hbm, out_hbm,
                  hist_sh, idxv0, idxv1, onesv, stagev, sem0, sem1):
    c = lax.axis_index("c")
    s = lax.axis_index("s")
    wid = s * SC_NC + c
    pltpu.sync_copy(zeros_hbm, stagev)
    pltpu.sync_copy(stagev, hist_sh.at[pl.ds(s * STRIPE, STRIPE)])
    pltpu.sync_copy(ones_hbm, onesv)
    plsc.subcore_barrier()
    base = wid * SPAN
    bufs = (idxv0, idxv1)
    sems = (sem0, sem1)
    cps = [None, None]
    cps[0] = pltpu.async_copy(idx_hbm.at[pl.ds(base, SC_CH)], idxv0, sem0)
    for j in range(SC_NCHUNK):
        nxt = (j + 1) % 2
        if j + 1 < SC_NCHUNK:
            cps[nxt] = pltpu.async_copy(
                idx_hbm.at[pl.ds(base + (j + 1) * SC_CH, SC_CH)],
                bufs[nxt], sems[nxt])
        cps[j % 2].wait()
        pltpu.sync_copy(onesv, hist_sh.at[bufs[j % 2]], add=True)
    plsc.subcore_barrier()
    pltpu.sync_copy(hist_sh.at[pl.ds(s * STRIPE, STRIPE)], stagev)
    pltpu.sync_copy(stagev, out_hbm.at[c, pl.ds(s * STRIPE, STRIPE)])


def _sc_histogram(idx_flat, zeros, ones):
    mesh = plsc.VectorSubcoreMesh(core_axis_name="c", subcore_axis_name="s")
    kern = functools.partial(
        pl.kernel,
        mesh=mesh,
        compiler_params=pltpu.CompilerParams(needs_layout_passes=False),
        out_type=jax.ShapeDtypeStruct((SC_NC, NB), jnp.float32),
        scratch_types=[
            pltpu.VMEM_SHARED((NB,), jnp.float32),
            pltpu.VMEM((SC_CH,), jnp.int32),
            pltpu.VMEM((SC_CH,), jnp.int32),
            pltpu.VMEM((SC_CH,), jnp.float32),
            pltpu.VMEM((STRIPE,), jnp.float32),
            pltpu.SemaphoreType.DMA,
            pltpu.SemaphoreType.DMA,
        ],
    )(_sc_hist_body)
    return kern(idx_flat, zeros, ones)


def _fin_body(hist_ref, stats_ref, sfg_ref, out_ref):
    hist = hist_ref[0] + hist_ref[1]
    ki = lax.broadcasted_iota(jnp.int32, (LANES, LANES), 0)
    ci = lax.broadcasted_iota(jnp.int32, (LANES, LANES), 1)
    t_suf = (ki >= ci).astype(jnp.float32)
    kr = lax.broadcasted_iota(jnp.int32, (KROWS, KROWS), 0)
    cr = lax.broadcasted_iota(jnp.int32, (KROWS, KROWS), 1)
    a_suf = (cr > kr).astype(jnp.float32)
    wbin = jnp.float32(2.0 / K)
    total = jnp.float32(0.0)
    for b in range(B):
        inst = jnp.float32(0.0)
        var = jnp.float32(0.0)
        seedl = stats_ref[b, 0, 56]
        obj = jnp.float32(0.0)
        for i in range(4):
            sl = b * 4 + i
            r0 = sl * 2 * KROWS
            n0 = hist[r0:r0 + KROWS, :]
            n1 = hist[r0 + KROWS:r0 + 2 * KROWS, :]
            tot = n0 + n1
            wc = jnp.dot(tot, t_suf, preferred_element_type=jnp.float32)
            wg = jnp.dot(n1, t_suf, preferred_element_type=jnp.float32)
            rc = jnp.dot(a_suf, wc[:, 0:1], preferred_element_type=jnp.float32)
            rg = jnp.dot(a_suf, wg[:, 0:1], preferred_element_type=jnp.float32)
            sc_ = wc + rc
            sg = wg + rg
            o = i * NSTAT
            cnt = stats_ref[b, 0, o + 0]
            present = (cnt > 0.0).astype(jnp.float32)
            safe = jnp.maximum(cnt, 1.0)
            g_tot = cnt
            h = (g_tot - sg) / jnp.maximum(g_tot + sc_ - sg, 1.0)
            h_k = g_tot / jnp.maximum(g_tot, 1.0)
            hsum = jnp.sum(h) + h_k
            s_lov = 2.0 - wbin * (hsum - 0.5 * h_k)
            inst = inst + present * s_lov
            vs = jnp.float32(0.0)
            for k in range(3):
                ssum = stats_ref[b, 0, o + 8 + k]
                s2sum = stats_ref[b, 0, o + 11 + k]
                sm = ssum / safe
                vs = vs + (s2sum - 2.0 * sm * ssum + sm * sm * cnt)
            var = var + present * vs / (3.0 * safe)
            seedl = seedl + present * 10.0 * sfg_ref[b, 0, i]
            obj = obj + present
        so = jnp.maximum(obj, 1.0)
        total = total + inst / so + 10.0 * var / so + seedl / jnp.float32(P)
    out_ref[0] = total * jnp.float32(1.0 / B)


def _finalize(hist, stats, sfg):
    return pl.pallas_call(
        _fin_body,
        grid=(1,),
        in_specs=[
            pl.BlockSpec((SC_NC, NB // LANES, LANES), lambda _: (0, 0, 0)),
            pl.BlockSpec((B, 1, 64), lambda _: (0, 0, 0),
                         memory_space=pltpu.SMEM),
            pl.BlockSpec((B, 1, 8), lambda _: (0, 0, 0),
                         memory_space=pltpu.SMEM),
        ],
        out_specs=pl.BlockSpec((1,), lambda _: (0,), memory_space=pltpu.SMEM),
        out_shape=jax.ShapeDtypeStruct((1,), jnp.float32),
    )(hist, stats, sfg)


def kernel(prediction, GT, CL, CE):
    pred = prediction.reshape(B, 7, ROWS, LANES)
    gt = GT.reshape(B, ROWS, LANES).astype(jnp.int32)
    cl = CL.reshape(B, ROWS, LANES).astype(jnp.int32)
    ce = CE.reshape(B, ROWS, LANES).astype(jnp.int32)

    stats = _pass1(pred, gt, cl, ce)
    binidx, sfg = _pass2(pred, gt, stats)
    zeros = jnp.zeros((STRIPE,), jnp.float32)
    ones = jnp.ones((SC_CH,), jnp.float32)
    hist = _sc_histogram(binidx.reshape(N_ELEM), zeros, ones)
    out = _finalize(hist.reshape(SC_NC, NB // LANES, LANES), stats, sfg)
    return out[0]


# speedup vs baseline: 1.5352x; 1.0617x over previous
import functools

import jax
import jax.numpy as jnp
from jax import lax
from jax.experimental import pallas as pl
from jax.experimental.pallas import tpu as pltpu
from jax.experimental.pallas import tpu_sc as plsc

B = 2
ROWS = 4096
LANES = 128
P = ROWS * LANES
CHR = 512
NCH = ROWS // CHR

K = 2048
NSLOT = 8
NBC = 2 * K
KROWS = K // LANES

SC_NC = 2
SC_NS = 16
SC_L = 16
NW = SC_NC * SC_NS
N_ELEM = 4 * P
SPAN = N_ELEM // NW
SC_CH = 16384
SC_NCHUNK = SPAN // SC_CH
NB = 4 * NBC
STRIPE = NB // SC_NS

FMAX = 3.4028235e38
NSTAT = 14


def _xyz(ch):
    ri = lax.broadcasted_iota(jnp.int32, (CHR, LANES), 0) + ch * CHR
    li = lax.broadcasted_iota(jnp.int32, (CHR, LANES), 1)
    z = ri // 128
    y = ri - z * 128
    xm = li.astype(jnp.float32) * (1.0 / 127.0)
    ym = y.astype(jnp.float32) * (1.0 / 127.0)
    zm = z.astype(jnp.float32) * (1.0 / 31.0)
    return xm, ym, zm


def _p1_body(sig_ref, seed_ref, gt_ref, cl_ref, ce_ref, out_ref, acc):
    ch = pl.program_id(1)

    @pl.when(ch == 0)
    def _init():
        for j in range(64):
            acc[j] = 0.0

    xm, ym, zm = _xyz(ch)
    gt = gt_ref[0]
    ce_f = (ce_ref[0] != 0).astype(jnp.float32)
    cl_f = (cl_ref[0] != 0).astype(jnp.float32)
    seed = jax.nn.sigmoid(seed_ref[0, 0])
    acc[56] = acc[56] + jnp.sum(seed * seed * (1.0 - cl_f))
    sx = sig_ref[0, 0]
    sy = sig_ref[0, 1]
    sz = sig_ref[0, 2]
    for i in range(4):
        mf = (gt == i + 1).astype(jnp.float32)
        cm = mf * ce_f
        o = i * NSTAT
        acc[o + 0] = acc[o + 0] + jnp.sum(mf)
        acc[o + 1] = acc[o + 1] + jnp.sum(cm)
        acc[o + 2] = acc[o + 2] + jnp.sum(xm * cm)
        acc[o + 3] = acc[o + 3] + jnp.sum(ym * cm)
        acc[o + 4] = acc[o + 4] + jnp.sum(zm * cm)
        acc[o + 5] = acc[o + 5] + jnp.sum(xm * mf)
        acc[o + 6] = acc[o + 6] + jnp.sum(ym * mf)
        acc[o + 7] = acc[o + 7] + jnp.sum(zm * mf)
        acc[o + 8] = acc[o + 8] + jnp.sum(sx * mf)
        acc[o + 9] = acc[o + 9] + jnp.sum(sy * mf)
        acc[o + 10] = acc[o + 10] + jnp.sum(sz * mf)
        acc[o + 11] = acc[o + 11] + jnp.sum(sx * sx * mf)
        acc[o + 12] = acc[o + 12] + jnp.sum(sy * sy * mf)
        acc[o + 13] = acc[o + 13] + jnp.sum(sz * sz * mf)

    @pl.when(ch == NCH - 1)
    def _flush():
        for j in range(64):
            out_ref[0, 0, j] = acc[j]


def _pass1(pred, gt, cl, ce):
    return pl.pallas_call(
        _p1_body,
        grid=(B, NCH),
        in_specs=[
            pl.BlockSpec((1, 3, CHR, LANES), lambda b, ch: (b, 1, ch, 0)),
            pl.BlockSpec((1, 1, CHR, LANES), lambda b, ch: (b, 6, ch, 0)),
            pl.BlockSpec((1, CHR, LANES), lambda b, ch: (b, ch, 0)),
            pl.BlockSpec((1, CHR, LANES), lambda b, ch: (b, ch, 0)),
            pl.BlockSpec((1, CHR, LANES), lambda b, ch: (b, ch, 0)),
        ],
        out_specs=pl.BlockSpec((1, 1, 64), lambda b, ch: (b, 0, 0),
                               memory_space=pltpu.SMEM),
        out_shape=jax.ShapeDtypeStruct((B, 1, 64), jnp.float32),
        scratch_shapes=[pltpu.SMEM((64,), jnp.float32)],
    )(pred, pred, gt, cl, ce)


def _p2_body(emb_ref, seed_ref, gt_ref, stats_ref, bin_ref, sfg_ref, acc):
    ch = pl.program_id(0)

    @pl.when(ch == 0)
    def _init():
        for j in range(8):
            acc[j] = 0.0

    xm, ym, zm = _xyz(ch)
    ex = jnp.tanh(emb_ref[0, 0]) + xm
    ey = jnp.tanh(emb_ref[0, 1]) + ym
    ez = jnp.tanh(emb_ref[0, 2]) + zm
    seed = jax.nn.sigmoid(seed_ref[0, 0])
    gt = gt_ref[0]
    kf = jnp.float32(K)
    for i in range(4):
        o = i * NSTAT
        cnt = stats_ref[0, 0, o + 0]
        ccnt = stats_ref[0, 0, o + 1]
        safe = jnp.maximum(cnt, 1.0)
        one_c = ccnt == 1.0
        cx = jnp.where(one_c, stats_ref[0, 0, o + 2], stats_ref[0, 0, o + 5] / safe)
        cy = jnp.where(one_c, stats_ref[0, 0, o + 3], stats_ref[0, 0, o + 6] / safe)
        cz = jnp.where(one_c, stats_ref[0, 0, o + 4], stats_ref[0, 0, o + 7] / safe)
        s4x = jnp.minimum(jnp.exp(10.0 * stats_ref[0, 0, o + 8] / safe), FMAX)
        s4y = jnp.minimum(jnp.exp(10.0 * stats_ref[0, 0, o + 9] / safe), FMAX)
        s4z = jnp.minimum(jnp.exp(10.0 * stats_ref[0, 0, o + 10] / safe), FMAX)
        q = ((ex - cx) * (ex - cx) * s4x + (ey - cy) * (ey - cy) * s4y
             + (ez - cz) * (ez - cz) * s4z)
        d = jnp.exp(-q)
        mi = gt == i + 1
        mf = mi.astype(jnp.float32)
        dv = seed - d
        acc[i] = acc[i] + jnp.sum(dv * dv * mf)
        binf = jnp.where(mi, kf - kf * d, kf * d)
        binn = jnp.clip(jnp.floor(binf).astype(jnp.int32), 0, K - 1)
        slotbase = (i * 2) * K
        bin_ref[i] = binn + jnp.where(mi, slotbase + K, slotbase)

    @pl.when(ch == NCH - 1)
    def _flush():
        for j in range(8):
            sfg_ref[0, j] = acc[j]


def _pass2(pred, gt, stats, b):
    return pl.pallas_call(
        _p2_body,
        grid=(NCH,),
        in_specs=[
            pl.BlockSpec((1, 3, CHR, LANES), lambda ch: (b, 0, ch, 0)),
            pl.BlockSpec((1, 1, CHR, LANES), lambda ch: (b, 6, ch, 0)),
            pl.BlockSpec((1, CHR, LANES), lambda ch: (b, ch, 0)),
            pl.BlockSpec((1, 1, 64), lambda ch: (b, 0, 0),
                         memory_space=pltpu.SMEM),
        ],
        out_specs=[
            pl.BlockSpec((4, CHR, LANES), lambda ch: (0, ch, 0)),
            pl.BlockSpec((1, 8), lambda ch: (0, 0),
                         memory_space=pltpu.SMEM),
        ],
        out_shape=[
            jax.ShapeDtypeStruct((4, ROWS, LANES), jnp.int32),
            jax.ShapeDtypeStruct((1, 8), jnp.float32),
        ],
        scratch_shapes=[pltpu.SMEM((8,), jnp.float32)],
    )(pred, pred, gt, stats)


def _sc_hist_body(idx_hbm, zeros_hbm, ones_hbm, out_hbm,
                  hist_sh, idxv0, idxv1, onesv, stagev, sem0, sem1):
    c = lax.axis_index("c")
    s = lax.axis_index("s")
    wid = s * SC_NC + c
    pltpu.sync_copy(zeros_hbm, stagev)
    pltpu.sync_copy(stagev, hist_sh.at[pl.ds(s * STRIPE, STRIPE)])
    pltpu.sync_copy(ones_hbm, onesv)
    plsc.subcore_barrier()
    base = wid * SPAN
    bufs = (idxv0, idxv1)
    sems = (sem0, sem1)
    cps = [None, None]
    cps[0] = pltpu.async_copy(idx_hbm.at[pl.ds(base, SC_CH)], idxv0, sem0)
    for j in range(SC_NCHUNK):
        nxt = (j + 1) % 2
        if j + 1 < SC_NCHUNK:
            cps[nxt] = pltpu.async_copy(
                idx_hbm.at[pl.ds(base + (j + 1) * SC_CH, SC_CH)],
                bufs[nxt], sems[nxt])
        cps[j % 2].wait()
        pltpu.sync_copy(onesv, hist_sh.at[bufs[j % 2]], add=True)
    plsc.subcore_barrier()
    pltpu.sync_copy(hist_sh.at[pl.ds(s * STRIPE, STRIPE)], stagev)
    pltpu.sync_copy(stagev, out_hbm.at[c, pl.ds(s * STRIPE, STRIPE)])


def _sc_histogram(idx_flat, zeros, ones):
    mesh = plsc.VectorSubcoreMesh(core_axis_name="c", subcore_axis_name="s")
    kern = functools.partial(
        pl.kernel,
        mesh=mesh,
        compiler_params=pltpu.CompilerParams(needs_layout_passes=False),
        out_type=jax.ShapeDtypeStruct((SC_NC, NB), jnp.float32),
        scratch_types=[
            pltpu.VMEM_SHARED((NB,), jnp.float32),
            pltpu.VMEM((SC_CH,), jnp.int32),
            pltpu.VMEM((SC_CH,), jnp.int32),
            pltpu.VMEM((SC_CH,), jnp.float32),
            pltpu.VMEM((STRIPE,), jnp.float32),
            pltpu.SemaphoreType.DMA,
            pltpu.SemaphoreType.DMA,
        ],
    )(_sc_hist_body)
    return kern(idx_flat, zeros, ones)


def _fin_body(hist0_ref, hist1_ref, stats_ref, sfg0_ref, sfg1_ref, out_ref):
    ki = lax.broadcasted_iota(jnp.int32, (LANES, LANES), 0)
    ci = lax.broadcasted_iota(jnp.int32, (LANES, LANES), 1)
    t_suf = (ki >= ci).astype(jnp.float32)
    kr = lax.broadcasted_iota(jnp.int32, (KROWS, KROWS), 0)
    cr = lax.broadcasted_iota(jnp.int32, (KROWS, KROWS), 1)
    a_suf = (cr > kr).astype(jnp.float32)
    wbin = jnp.float32(2.0 / K)
    total = jnp.float32(0.0)
    for b in range(B):
        hr = hist0_ref if b == 0 else hist1_ref
        hist = hr[0] + hr[1]
        sfgr = sfg0_ref if b == 0 else sfg1_ref
        inst = jnp.float32(0.0)
        var = jnp.float32(0.0)
        seedl = stats_ref[b, 0, 56]
        obj = jnp.float32(0.0)
        for i in range(4):
            r0 = i * 2 * KROWS
            n0 = hist[r0:r0 + KROWS, :]
            n1 = hist[r0 + KROWS:r0 + 2 * KROWS, :]
            tot = n0 + n1
            wc = jnp.dot(tot, t_suf, preferred_element_type=jnp.float32)
            wg = jnp.dot(n1, t_suf, preferred_element_type=jnp.float32)
            rc = jnp.dot(a_suf, wc[:, 0:1], preferred_element_type=jnp.float32)
            rg = jnp.dot(a_suf, wg[:, 0:1], preferred_element_type=jnp.float32)
            sc_ = wc + rc
            sg = wg + rg
            o = i * NSTAT
            cnt = stats_ref[b, 0, o + 0]
            present = (cnt > 0.0).astype(jnp.float32)
            safe = jnp.maximum(cnt, 1.0)
            g_tot = cnt
            h = (g_tot - sg) / jnp.maximum(g_tot + sc_ - sg, 1.0)
            h_k = g_tot / jnp.maximum(g_tot, 1.0)
            hsum = jnp.sum(h) + h_k
            s_lov = 2.0 - wbin * (hsum - 0.5 * h_k)
            inst = inst + present * s_lov
            vs = jnp.float32(0.0)
            for k in range(3):
                ssum = stats_ref[b, 0, o + 8 + k]
                s2sum = stats_ref[b, 0, o + 11 + k]
                sm = ssum / safe
                vs = vs + (s2sum - 2.0 * sm * ssum + sm * sm * cnt)
            var = var + present * vs / (3.0 * safe)
            seedl = seedl + present * 10.0 * sfgr[0, i]
            obj = obj + present
        so = jnp.maximum(obj, 1.0)
        total = total + inst / so + 10.0 * var / so + seedl / jnp.float32(P)
    out_ref[0] = total * jnp.float32(1.0 / B)


def _finalize(hist0, hist1, stats, sfg0, sfg1):
    return pl.pallas_call(
        _fin_body,
        grid=(1,),
        in_specs=[
            pl.BlockSpec((SC_NC, NB // LANES, LANES), lambda _: (0, 0, 0)),
            pl.BlockSpec((SC_NC, NB // LANES, LANES), lambda _: (0, 0, 0)),
            pl.BlockSpec((B, 1, 64), lambda _: (0, 0, 0),
                         memory_space=pltpu.SMEM),
            pl.BlockSpec((1, 8), lambda _: (0, 0), memory_space=pltpu.SMEM),
            pl.BlockSpec((1, 8), lambda _: (0, 0), memory_space=pltpu.SMEM),
        ],
        out_specs=pl.BlockSpec((1,), lambda _: (0,), memory_space=pltpu.SMEM),
        out_shape=jax.ShapeDtypeStruct((1,), jnp.float32),
    )(hist0, hist1, stats, sfg0, sfg1)


def kernel(prediction, GT, CL, CE):
    pred = prediction.reshape(B, 7, ROWS, LANES)
    gt = GT.reshape(B, ROWS, LANES).astype(jnp.int32)
    cl = CL.reshape(B, ROWS, LANES).astype(jnp.int32)
    ce = CE.reshape(B, ROWS, LANES).astype(jnp.int32)

    stats = _pass1(pred, gt, cl, ce)
    zeros = jnp.zeros((STRIPE,), jnp.float32)
    ones = jnp.ones((SC_CH,), jnp.float32)
    binidx0, sfg0 = _pass2(pred, gt, stats, 0)
    hist0 = _sc_histogram(binidx0.reshape(N_ELEM), zeros, ones)
    binidx1, sfg1 = _pass2(pred, gt, stats, 1)
    hist1 = _sc_histogram(binidx1.reshape(N_ELEM), zeros, ones)
    out = _finalize(hist0.reshape(SC_NC, NB // LANES, LANES),
                    hist1.reshape(SC_NC, NB // LANES, LANES),
                    stats, sfg0, sfg1)
    return out[0]
